# baseline (speedup 1.0000x reference)
import functools

import jax
import jax.numpy as jnp
from jax import lax
from jax.experimental import pallas as pl
from jax.experimental.pallas import tpu as pltpu
from jax.experimental.pallas import tpu_sc as plsc

D = 64
ROWS = 256
NB = 4
AHEAD = NB - 1


def _build(N, NC, NS):
    NW = NC * NS
    per_w = N // NW
    rows_per_w = per_w // ROWS
    outer = rows_per_w
    assert per_w * NW == N and rows_per_w * ROWS == per_w and outer > NB

    mesh = plsc.VectorSubcoreMesh(core_axis_name="c", subcore_axis_name="s")

    bufs = [pltpu.VMEM((ROWS, D), jnp.float32) for _ in range(NB)]
    gsems = [pltpu.SemaphoreType.DMA for _ in range(NB)]
    osems = [pltpu.SemaphoreType.DMA for _ in range(NB)]

    @functools.partial(
        pl.kernel,
        out_type=jax.ShapeDtypeStruct((N, D), jnp.float32),
        mesh=mesh,
        compiler_params=pltpu.CompilerParams(use_tc_tiling_on_sc=False),
        scratch_types=[pltpu.VMEM((rows_per_w, ROWS), jnp.int32)]
        + bufs + gsems + osems,
    )
    def gather_kernel(table_hbm, idx_hbm, out_hbm, idx_v, *scratch):
        rows = scratch[:NB]
        gsem = scratch[NB:2 * NB]
        osem = scratch[2 * NB:3 * NB]

        wid = lax.axis_index("s") * NC + lax.axis_index("c")
        out_base = wid * per_w

        pltpu.sync_copy(idx_hbm.at[pl.ds(wid * rows_per_w, rows_per_w)], idx_v)

        def fire(b, s):
            pltpu.make_async_copy(
                table_hbm.at[idx_v.at[s]], rows[b], gsem[b]).start()

        def wait_gather(b):
            pltpu.make_async_copy(
                out_hbm.at[pl.ds(0, ROWS)], rows[b], gsem[b]).wait()

        def flush(b, s):
            pltpu.make_async_copy(
                rows[b], out_hbm.at[pl.ds(out_base + s * ROWS, ROWS)],
                osem[b]).start()

        def wait_flush(b):
            pltpu.make_async_copy(
                rows[b], out_hbm.at[pl.ds(out_base, ROWS)], osem[b]).wait()

        for b in range(AHEAD):
            fire(b, b)

        def body(o, carry):
            for b in range(NB):
                s = NB * o + b
                nb = (b + AHEAD) % NB

                @pl.when(s + AHEAD < outer)
                def _():
                    @pl.when(s + AHEAD >= NB)
                    def _():
                        wait_flush(nb)
                    fire(nb, s + AHEAD)

                wait_gather(b)
                flush(b, s)
            return carry

        lax.fori_loop(0, outer // NB, body, 0)
        for b in range(NB):
            wait_flush(b)

    return gather_kernel


def kernel(input, indices):
    B, K = indices.shape
    N = B * K
    info = plsc.get_sparse_core_info()
    NC, NS = info.num_cores, info.num_subcores
    idx2d = indices.reshape(N // ROWS, ROWS).astype(jnp.int32)
    out = _build(N, NC, NS)(input, idx2d)
    return out.reshape(B, K, D)
